# baseline (speedup 1.0000x reference)
import functools

import jax
import jax.numpy as jnp
from jax import lax
from jax.experimental import pallas as pl
from jax.experimental.pallas import tpu as pltpu
from jax.experimental.pallas import tpu_sc as plsc

N = 10000
E = 320000
IN_DIM = 128
H = 256
L = 4
NC = 10

NCORE = 2
NSUB = 16
N_PAD = 10240
EB = 128
NBLK = 168
BK = NBLK * EB
E_PAD = NSUB * BK
ROWS_T = N_PAD // NSUB
AROWS = 648

_INTERPRET = False
RBLK = 256
RBLK_E = 512


def _embed_body(h_ref, w_ref, b_ref, o_ref):
    acc = jnp.dot(h_ref[...], w_ref[...], preferred_element_type=jnp.float32)
    acc = acc + b_ref[0:1, :]
    for c in range(4):
        o_ref[c] = acc[:, c * 64:(c + 1) * 64]


def _embed(h_pad, Wh, bh_t):
    R = RBLK
    return pl.pallas_call(
        _embed_body,
        grid=(N_PAD // R,),
        in_specs=[
            pl.BlockSpec((R, IN_DIM), lambda i: (i, 0)),
            pl.BlockSpec((IN_DIM, H), lambda i: (0, 0)),
            pl.BlockSpec((8, H), lambda i: (0, 0)),
        ],
        out_specs=pl.BlockSpec((4, R, 64), lambda i: (0, i, 0)),
        out_shape=jax.ShapeDtypeStruct((4, N_PAD, 64), jnp.float32),
        interpret=_INTERPRET,
    )(h_pad, Wh, bh_t)


def _tables_body(hh_ref, w_ref, b_ref, a_ref, ts_ref, td_ref):
    R = hh_ref.shape[1]
    acc = jnp.zeros((R, 4 * H), jnp.float32)
    for c in range(4):
        acc = acc + jnp.dot(hh_ref[c], w_ref[pl.ds(c * 64, 64), :],
                            preferred_element_type=jnp.float32)
    acc = acc + b_ref[0:1, :]
    for c in range(4):
        a_ref[c] = acc[:, c * 64:(c + 1) * 64]
        d_c = acc[:, 512 + c * 64:512 + (c + 1) * 64]
        b_c = acc[:, 256 + c * 64:256 + (c + 1) * 64]
        ts_ref[c] = jnp.concatenate([d_c, b_c], axis=1)
        td_ref[c] = acc[:, 768 + c * 64:768 + (c + 1) * 64]


def _tables(hh, Wcat, bcat_t):
    R = RBLK
    return pl.pallas_call(
        _tables_body,
        grid=(N_PAD // R,),
        in_specs=[
            pl.BlockSpec((4, R, 64), lambda i: (0, i, 0)),
            pl.BlockSpec((H, 4 * H), lambda i: (0, 0)),
            pl.BlockSpec((8, 4 * H), lambda i: (0, 0)),
        ],
        out_specs=[
            pl.BlockSpec((4, R, 64), lambda i: (0, i, 0)),
            pl.BlockSpec((4, R, 128), lambda i: (0, i, 0)),
            pl.BlockSpec((4, R, 64), lambda i: (0, i, 0)),
        ],
        out_shape=[
            jax.ShapeDtypeStruct((4, N_PAD, 64), jnp.float32),
            jax.ShapeDtypeStruct((4, N_PAD, 128), jnp.float32),
            jax.ShapeDtypeStruct((4, N_PAD, 64), jnp.float32),
        ],
        interpret=_INTERPRET,
    )(hh, Wcat, bcat_t)


def _ce_body(nr, *refs):
    r_refs = refs[:nr]
    w_refs = refs[nr:2 * nr]
    o_ref = refs[2 * nr]
    R = r_refs[0].shape[1]
    acc = jnp.zeros((R, H), jnp.float32)
    for j in range(nr):
        for c in range(4):
            acc = acc + jnp.dot(r_refs[j][c], w_refs[j][pl.ds(c * 64, 64), :],
                                preferred_element_type=jnp.float32)
    for c in range(4):
        o_ref[c] = acc[:, c * 64:(c + 1) * 64]


def _ce_matmul(rs, Wfs):
    nr = len(rs)
    R = RBLK_E
    return pl.pallas_call(
        functools.partial(_ce_body, nr),
        grid=(E_PAD // R,),
        in_specs=[pl.BlockSpec((4, R, 64), lambda i: (0, i, 0)) for _ in range(nr)]
        + [pl.BlockSpec((H, H), lambda i: (0, 0)) for _ in range(nr)],
        out_specs=pl.BlockSpec((4, R, 64), lambda i: (0, i, 0)),
        out_shape=jax.ShapeDtypeStruct((4, E_PAD, 64), jnp.float32),
        interpret=_INTERPRET,
    )(*rs, *Wfs)


def _hupd_body(a_ref, nd_ref, hh_ref, g_ref, b_ref, o_ref):
    a = a_ref[0]
    ndv = nd_ref[0]
    hhv = hh_ref[0]
    num = ndv[:, :64]
    den = ndv[:, 64:] + 1e-6
    t = jnp.maximum(a + num / den, 0.0)
    tv = t[:N, :]
    s1 = jnp.sum(tv, axis=0)
    s2 = jnp.sum(tv * tv, axis=0)
    m = s1 / N
    v = s2 / N - m * m
    scale = g_ref[0, 0] * lax.rsqrt(v + 1e-5)
    shift = b_ref[0, 0] - scale * m
    o_ref[0] = hhv + t * scale[None, :] + shift[None, :]


def _hupd(A, nd, hh, g4, b4):
    return pl.pallas_call(
        _hupd_body,
        grid=(4,),
        in_specs=[
            pl.BlockSpec((1, N_PAD, 64), lambda c: (c, 0, 0)),
            pl.BlockSpec((1, N_PAD, 128), lambda c: (c, 0, 0)),
            pl.BlockSpec((1, N_PAD, 64), lambda c: (c, 0, 0)),
            pl.BlockSpec((1, 1, 64), lambda c: (c, 0, 0)),
            pl.BlockSpec((1, 1, 64), lambda c: (c, 0, 0)),
        ],
        out_specs=pl.BlockSpec((1, N_PAD, 64), lambda c: (c, 0, 0)),
        out_shape=jax.ShapeDtypeStruct((4, N_PAD, 64), jnp.float32),
        interpret=_INTERPRET,
    )(A, nd, hh, g4, b4)


def _final_body(hh_ref, w_ref, b_ref, o_ref):
    R = hh_ref.shape[1]
    acc = jnp.zeros((R, 128), jnp.float32)
    for c in range(4):
        acc = acc + jnp.dot(hh_ref[c], w_ref[pl.ds(c * 64, 64), :],
                            preferred_element_type=jnp.float32)
    o_ref[...] = acc + b_ref[0:1, :]


def _final(hh, Woutp, bout_t):
    R = RBLK
    return pl.pallas_call(
        _final_body,
        grid=(N_PAD // R,),
        in_specs=[
            pl.BlockSpec((4, R, 64), lambda i: (0, i, 0)),
            pl.BlockSpec((H, 128), lambda i: (0, 0)),
            pl.BlockSpec((8, 128), lambda i: (0, 0)),
        ],
        out_specs=pl.BlockSpec((R, 128), lambda i: (i, 0)),
        out_shape=jax.ShapeDtypeStruct((N_PAD, 128), jnp.float32),
        interpret=_INTERPRET,
    )(hh, Woutp, bout_t)


def _sc_edge_call(has_ce, has_r, ce_flat, tsrc, tdst, srcs, dsts, ers, uc, zrow):
    mesh = plsc.VectorSubcoreMesh(core_axis_name="c", subcore_axis_name="s",
                                  num_cores=NCORE, num_subcores=NSUB)
    out_type = []
    if has_r:
        out_type.append(jax.ShapeDtypeStruct((4 * E_PAD, 64), jnp.float32))
    out_type.append(jax.ShapeDtypeStruct((4, N_PAD, 128), jnp.float32))
    if has_r:
        out_type.append(jax.ShapeDtypeStruct((64, 2, 64), jnp.float32))

    scratch = [
        pltpu.VMEM((EB,), jnp.int32),
        pltpu.VMEM((EB,), jnp.int32),
        pltpu.VMEM((EB,), jnp.float32),
        pltpu.VMEM((EB,), jnp.int32),
        pltpu.VMEM((EB,), jnp.int32),
        pltpu.VMEM((EB, 128), jnp.float32),
        pltpu.VMEM((EB, 64), jnp.float32),
        pltpu.VMEM((EB, 64), jnp.float32),
        pltpu.VMEM((EB, 64), jnp.float32),
        pltpu.VMEM((2, 64), jnp.float32),
        pltpu.VMEM((2, 64), jnp.float32),
        pltpu.VMEM((AROWS, 128), jnp.float32),
    ]

    def body(*refs):
        ins = list(refs)
        if has_ce:
            ce_r = ins.pop(0)
        tsrc_r, tdst_r, srcs_r, dsts_r, ers_r, uc_r, zrow_r = ins[:7]
        ins = ins[7:]
        if has_r:
            r_out, nd_out, bn_out = ins[:3]
            ins = ins[3:]
        else:
            nd_out = ins.pop(0)
        (srcv, dstv, erv, sa, da, tsg, tdg, ceb, rb, ucb, bnv, acc) = ins

        cid = lax.axis_index("c")
        sid = lax.axis_index("s")

        for t in range(2):
            q = 2 * t + cid
            qn = q * N_PAD
            qe = q * E_PAD + sid * BK
            pltpu.sync_copy(uc_r.at[q], ucb)
            uvec = [ucb[0, pl.ds(j * 16, 16)] for j in range(4)]
            cvec = [ucb[1, pl.ds(j * 16, 16)] for j in range(4)]
            pltpu.sync_copy(zrow_r, acc)

            def blk_body(b, bn_carry):
                pltpu.sync_copy(srcs_r.at[sid, b], srcv)
                pltpu.sync_copy(dsts_r.at[sid, b], dstv)
                pltpu.sync_copy(ers_r.at[sid, b], erv)
                base = sid * ROWS_T + qn
                for v in range(EB // 16):
                    sl = pl.ds(v * 16, 16)
                    sa[sl] = srcv[sl] + qn
                    da[sl] = jnp.minimum(dstv[sl], ROWS_T - 1) + base
                pltpu.sync_copy(tsrc_r.at[sa], tsg)
                pltpu.sync_copy(tdst_r.at[da], tdg)
                if has_ce:
                    pltpu.sync_copy(ce_r.at[pl.ds(qe + b * EB, EB)], ceb)

                def grp_body(g, carry):
                    bs0, bs1, bs2, bs3, bq0, bq1, bq2, bq3 = carry
                    bs = [bs0, bs1, bs2, bs3]
                    bq = [bq0, bq1, bq2, bq3]
                    er16 = erv[pl.ds(g * 16, 16)]
                    dl16 = dstv[pl.ds(g * 16, 16)]
                    for k2 in range(16):
                        k = g * 16 + k2
                        dl = dl16[k2]
                        eru = er16[k2] * jnp.ones((16,), jnp.float32)
                        if has_r:
                            bvf = jnp.where(dl < ROWS_T, 1.0, 0.0)
                            bv16 = bvf * jnp.ones((16,), jnp.float32)
                        for j in range(4):
                            sl = pl.ds(j * 16, 16)
                            sl2 = pl.ds(64 + j * 16, 16)
                            x = (tsg[k, sl] + tdg[k, sl] + eru * uvec[j]
                                 + cvec[j])
                            if has_ce:
                                x = x + ceb[k, sl]
                            sg = 1.0 / (1.0 + jnp.exp(-x))
                            acc[dl, sl] = acc[dl, sl] + sg * tsg[k, sl2]
                            acc[dl, sl2] = acc[dl, sl2] + sg
                            if has_r:
                                r = jnp.maximum(x, 0.0)
                                rb[k, sl] = r
                                rv = r * bv16
                                bs[j] = bs[j] + rv
                                bq[j] = bq[j] + rv * r
                    return (*bs, *bq)

                bn_carry = lax.fori_loop(0, EB // 16, grp_body, bn_carry)
                if has_r:
                    pltpu.sync_copy(rb, r_out.at[pl.ds(qe + b * EB, EB)])
                return bn_carry

            z = jnp.zeros((16,), jnp.float32)
            bn_carry = lax.fori_loop(0, NBLK, blk_body, (z,) * 8)
            pltpu.sync_copy(acc.at[pl.ds(0, ROWS_T)],
                            nd_out.at[q, pl.ds(sid * ROWS_T, ROWS_T)])
            if has_r:
                for j in range(4):
                    sl = pl.ds(j * 16, 16)
                    bnv[0, sl] = bn_carry[j]
                    bnv[1, sl] = bn_carry[4 + j]
                pltpu.sync_copy(bnv, bn_out.at[q * 16 + sid])

    ins = []
    if has_ce:
        ins.append(ce_flat)
    ins += [tsrc, tdst, srcs, dsts, ers, uc, zrow]
    k = pl.kernel(body, out_type=out_type, mesh=mesh, scratch_types=scratch,
                  compiler_params=pltpu.CompilerParams(
                      use_tc_tiling_on_sc=False),
                  interpret=_INTERPRET)
    outs = k(*ins)
    if has_r:
        return outs[0], outs[1], outs[2]
    return None, outs[0], None


def _tile8(v):
    return jnp.tile(v[None, :], (8, 1))


def kernel(h, edge_index, e, Wh, bh, We, be, layW, layb, gh, bh_bn, ge, be_bn,
           Wout, bout):
    f32 = jnp.float32
    h_pad = jnp.zeros((N_PAD, IN_DIM), f32).at[:N].set(h)
    src = edge_index[0]
    dst = edge_index[1]
    bucket = dst // ROWS_T
    order = jnp.argsort(bucket, stable=True)
    src_s = src[order]
    dst_s = dst[order]
    er_s = e[:, 0][order]
    bkt_s = bucket[order]
    starts = jnp.searchsorted(bkt_s, jnp.arange(NSUB, dtype=bkt_s.dtype))
    dest = bkt_s * BK + (jnp.arange(E) - starts[bkt_s])
    src_p = jnp.full((E_PAD,), N, jnp.int32).at[dest].set(src_s
                     ).reshape(NSUB, NBLK, EB)
    dst_p = jnp.full((E_PAD,), AROWS - 1, jnp.int32).at[dest].set(
        dst_s - bkt_s * ROWS_T).reshape(NSUB, NBLK, EB)
    er_p = jnp.zeros((E_PAD,), f32).at[dest].set(er_s).reshape(NSUB, NBLK, EB)
    zrow = jnp.zeros((AROWS, 128), f32)

    hh = _embed(h_pad, Wh, _tile8(bh))

    rs = []
    scales = []
    shift_sum = jnp.zeros((H,), f32)

    for i in range(L):
        Wcat = jnp.concatenate([layW[i, 0], layW[i, 1], layW[i, 3],
                                layW[i, 4]], axis=1)
        bcat = jnp.concatenate([layb[i, 0], layb[i, 1], layb[i, 3],
                                layb[i, 4]])
        A, tsrc, tdst = _tables(hh, Wcat, _tile8(bcat))

        Wi = layW[i, 2]
        u = (We @ Wi)[0]
        const = (be + shift_sum) @ Wi + layb[i, 2]
        uc = jnp.stack([u.reshape(4, 64), const.reshape(4, 64)], axis=1)

        if i > 0:
            Wfs = [s[:, None] * Wi for s in scales]
            ce = _ce_matmul(rs, Wfs).reshape(4 * E_PAD, 64)
        else:
            ce = None

        has_r = i < L - 1
        r_i, nd, bn = _sc_edge_call(
            i > 0, has_r, ce,
            tsrc.reshape(4 * N_PAD, 128), tdst.reshape(4 * N_PAD, 64),
            src_p, dst_p, er_p, uc, zrow)

        hh = _hupd(A, nd, hh, gh[i].reshape(4, 1, 64),
                   bh_bn[i].reshape(4, 1, 64))

        if has_r:
            bnr = bn.reshape(4, 16, 2, 64).sum(axis=1)
            s1 = bnr[:, 0, :].reshape(H)
            s2 = bnr[:, 1, :].reshape(H)
            m = s1 / E
            v = s2 / E - m * m
            sc = ge[i] / jnp.sqrt(v + 1e-5)
            sh = be_bn[i] - sc * m
            rs.append(r_i.reshape(4, E_PAD, 64))
            scales.append(sc)
            shift_sum = shift_sum + sh

    Woutp = jnp.zeros((H, 128), f32).at[:, :NC].set(Wout)
    out = _final(hh, Woutp, _tile8(jnp.zeros((128,), f32).at[:NC].set(bout)))
    return out[:N, :NC]
